# baseline (speedup 1.0000x reference)
import functools

import jax
import jax.numpy as jnp
from jax import lax
from jax.experimental import pallas as pl
from jax.experimental.pallas import tpu as pltpu
from jax.experimental.pallas import tpu_sc as plsc

NC = 2
NS = 16
NW = NC * NS
L = 16

N_NODES = 100000
D = 16
B = 16384
MOTIF = 3

BPW = B // NW
CHUNK = 128
NCH = BPW // CHUNK
GROUPS = BPW // L

FULL_TC = N_NODES // 128
TAIL0 = FULL_TC * 128
LIN_ROWS = 12504
COLS_PER_W = -(-FULL_TC // NW)


def _detile_body(emb_hbm, tail_hbm, out_hbm, buf_v, lin_v, tail_v):
    wid = lax.axis_index("s") * NC + lax.axis_index("c")
    lane = lax.iota(jnp.int32, L)

    def one_col(k, carry):
        c = k * NW + wid

        @pl.when(c < FULL_TC)
        def _():
            pltpu.sync_copy(emb_hbm.at[pl.ds(0, 8), pl.ds(128 * c, 128)],
                            buf_v.at[pl.ds(0, 8)])
            pltpu.sync_copy(emb_hbm.at[pl.ds(8, 8), pl.ds(128 * c, 128)],
                            buf_v.at[pl.ds(8, 8)])
            for l in range(128):
                row = plsc.load_gather(
                    buf_v, [lane, jnp.full((L,), l, jnp.int32)])
                lin_v[l // 8, pl.ds(16 * (l % 8), 16)] = row
            pltpu.sync_copy(lin_v, out_hbm.at[pl.ds(16 * c, 16)])

        return carry

    lax.fori_loop(0, COLS_PER_W, one_col, 0)

    @pl.when(wid == 0)
    def _():
        pltpu.sync_copy(tail_hbm, tail_v)
        pltpu.sync_copy(tail_v, out_hbm.at[pl.ds(LIN_ROWS - 8, 8)])


@functools.partial(
    pl.kernel,
    out_type=jax.ShapeDtypeStruct((LIN_ROWS, 128), jnp.float32),
    mesh=plsc.VectorSubcoreMesh(core_axis_name="c", subcore_axis_name="s"),
    compiler_params=pltpu.CompilerParams(needs_layout_passes=False,
                                         use_tc_tiling_on_sc=True),
    scratch_types=[
        pltpu.VMEM((16, 128), jnp.float32),
        pltpu.VMEM((16, 128), jnp.float32),
        pltpu.VMEM((8, 128), jnp.float32),
    ],
)
def _detile_sc(emb_hbm, tail_hbm, out_hbm, buf_v, lin_v, tail_v):
    _detile_body(emb_hbm, tail_hbm, out_hbm, buf_v, lin_v, tail_v)


def _sc_body(motifs_hbm, label_hbm, table_hbm, reward_hbm, parts_hbm,
             idx_v, rows_v, label_v, reward_v, tmat_v, parts_v, sem):
    wid = lax.axis_index("s") * NC + lax.axis_index("c")

    pltpu.sync_copy(motifs_hbm.at[:, wid], idx_v)
    pltpu.sync_copy(label_hbm.at[wid], label_v)

    copies = []
    for m in range(MOTIF):
        for j in range(NCH):
            copies.append(pltpu.async_copy(
                table_hbm.at[idx_v.at[m, j]],
                rows_v.at[pl.ds((m * NCH + j) * CHUNK, CHUNK)],
                sem))
    for c in copies:
        c.wait()

    lane = lax.iota(jnp.int32, L)
    tr_base = lane * L

    def group(g, acc):
        e0 = g * L
        for j in range(L):
            e = e0 + j
            prod = rows_v[e] * rows_v[BPW + e] * rows_v[2 * BPW + e]
            plsc.store_scatter(tmat_v, [tr_base + j], prod)
        score = tmat_v[pl.ds(0, L)]
        for d in range(1, D):
            score = score + tmat_v[pl.ds(d * L, L)]
        p = jnp.clip(1.0 - jnp.exp(-score), 1e-05, 1.0)
        reward_v[pl.ds(e0, L)] = 1.0 - p
        lbl = label_v[pl.ds(e0, L)]
        return acc + (lbl * p + (1.0 - lbl) * (1.0 - p))

    acc = lax.fori_loop(0, GROUPS, group, jnp.zeros((L,), jnp.float32))

    parts_v[...] = acc
    pltpu.sync_copy(reward_v, reward_hbm.at[wid])
    pltpu.sync_copy(parts_v, parts_hbm.at[wid])


@functools.partial(
    pl.kernel,
    out_type=[jax.ShapeDtypeStruct((NW, BPW), jnp.float32),
              jax.ShapeDtypeStruct((NW, L), jnp.float32)],
    mesh=plsc.VectorSubcoreMesh(core_axis_name="c", subcore_axis_name="s"),
    compiler_params=pltpu.CompilerParams(needs_layout_passes=False,
                                         use_tc_tiling_on_sc=False),
    scratch_types=[
        pltpu.VMEM((MOTIF, NCH, CHUNK), jnp.int32),
        pltpu.VMEM((MOTIF * BPW, D), jnp.float32),
        pltpu.VMEM((BPW,), jnp.float32),
        pltpu.VMEM((BPW,), jnp.float32),
        pltpu.VMEM((D * L,), jnp.float32),
        pltpu.VMEM((L,), jnp.float32),
        pltpu.SemaphoreType.DMA,
    ],
)
def _sc_kernel(motifs_hbm, label_hbm, table_hbm, reward_hbm, parts_hbm,
               idx_v, rows_v, label_v, reward_v, tmat_v, parts_v, sem):
    _sc_body(motifs_hbm, label_hbm, table_hbm, reward_hbm, parts_hbm,
             idx_v, rows_v, label_v, reward_v, tmat_v, parts_v, sem)


def _loss_body(parts_ref, out_ref):
    out_ref[0, 0] = -jnp.sum(parts_ref[...])


def _loss_finish(parts):
    return pl.pallas_call(
        _loss_body,
        out_shape=jax.ShapeDtypeStruct((1, 1), jnp.float32),
        out_specs=pl.BlockSpec(memory_space=pltpu.SMEM),
    )(parts)


@jax.jit
def kernel(embedding_matrix, motifs, label):
    tail = jnp.zeros((8 * 8, D), jnp.float32)
    tail = tail.at[:N_NODES - TAIL0].set(embedding_matrix[TAIL0:])
    lin = _detile_sc(embedding_matrix.T, tail.reshape(8, 128))
    table = lin.reshape(LIN_ROWS * 8, D)
    motifs_w = motifs.T.astype(jnp.int32).reshape(MOTIF, NW, NCH, CHUNK)
    label_w = label.reshape(NW, BPW)
    reward_w, parts = _sc_kernel(motifs_w, label_w, table)
    loss = _loss_finish(parts)[0, 0]
    return (loss, reward_w.reshape(B))


# speedup vs baseline: 1.1373x; 1.1373x over previous
import functools

import jax
import jax.numpy as jnp
from jax import lax
from jax.experimental import pallas as pl
from jax.experimental.pallas import tpu as pltpu
from jax.experimental.pallas import tpu_sc as plsc

NC = 2
NS = 16
NW = NC * NS
L = 16

N_NODES = 100000
D = 16
B = 16384
MOTIF = 3

BPW = B // NW
CHUNK = 128
NCH = BPW // CHUNK
GROUPS = BPW // L

FULL_TC = N_NODES // 128
TAIL0 = FULL_TC * 128
LIN_ROWS = 12504
CPW = 25
CLAST = FULL_TC - 31 * CPW


def _transpose_cols(buf_v, lin_v, lane, ncols):
    def one_col(k, carry):
        base = k * 128

        for l in range(128):
            row = plsc.load_gather(
                buf_v, [lane, jnp.full((L,), l, jnp.int32) + base])
            lin_v[k * 16 + l // 8, pl.ds(16 * (l % 8), 16)] = row
        return carry

    lax.fori_loop(0, ncols, one_col, 0)


def _detile_body(emb_hbm, tail_hbm, out_hbm, buf_v, lin_v, tail_v):
    wid = lax.axis_index("s") * NC + lax.axis_index("c")
    lane = lax.iota(jnp.int32, L)
    col0 = wid * (CPW * 128)

    @pl.when(wid < 31)
    def _():
        w = CPW * 128
        pltpu.sync_copy(emb_hbm.at[pl.ds(0, 8), pl.ds(col0, w)],
                        buf_v.at[pl.ds(0, 8)])
        pltpu.sync_copy(emb_hbm.at[pl.ds(8, 8), pl.ds(col0, w)],
                        buf_v.at[pl.ds(8, 8)])
        _transpose_cols(buf_v, lin_v, lane, CPW)
        pltpu.sync_copy(lin_v, out_hbm.at[pl.ds(wid * (CPW * 16), CPW * 16)])

    @pl.when(wid == 31)
    def _():
        w = CLAST * 128
        pltpu.sync_copy(emb_hbm.at[pl.ds(0, 8), pl.ds(col0, w)],
                        buf_v.at[pl.ds(0, 8), pl.ds(0, w)])
        pltpu.sync_copy(emb_hbm.at[pl.ds(8, 8), pl.ds(col0, w)],
                        buf_v.at[pl.ds(8, 8), pl.ds(0, w)])
        _transpose_cols(buf_v, lin_v, lane, CLAST)
        pltpu.sync_copy(lin_v.at[pl.ds(0, CLAST * 16)],
                        out_hbm.at[pl.ds(31 * (CPW * 16), CLAST * 16)])
        pltpu.sync_copy(tail_hbm, tail_v)
        pltpu.sync_copy(tail_v, out_hbm.at[pl.ds(LIN_ROWS - 8, 8)])


@functools.partial(
    pl.kernel,
    out_type=jax.ShapeDtypeStruct((LIN_ROWS, 128), jnp.float32),
    mesh=plsc.VectorSubcoreMesh(core_axis_name="c", subcore_axis_name="s"),
    compiler_params=pltpu.CompilerParams(needs_layout_passes=False,
                                         use_tc_tiling_on_sc=True),
    scratch_types=[
        pltpu.VMEM((16, CPW * 128), jnp.float32),
        pltpu.VMEM((CPW * 16, 128), jnp.float32),
        pltpu.VMEM((8, 128), jnp.float32),
    ],
)
def _detile_sc(emb_hbm, tail_hbm, out_hbm, buf_v, lin_v, tail_v):
    _detile_body(emb_hbm, tail_hbm, out_hbm, buf_v, lin_v, tail_v)


def _sc_body(motifs_hbm, label_hbm, table_hbm, reward_hbm, parts_hbm,
             idx_v, rows_v, label_v, reward_v, tmat_v, parts_v, sem):
    wid = lax.axis_index("s") * NC + lax.axis_index("c")

    pltpu.sync_copy(motifs_hbm.at[:, wid], idx_v)
    pltpu.sync_copy(label_hbm.at[wid], label_v)

    copies = []
    for m in range(MOTIF):
        for j in range(NCH):
            copies.append(pltpu.async_copy(
                table_hbm.at[idx_v.at[m, j]],
                rows_v.at[pl.ds((m * NCH + j) * CHUNK, CHUNK)],
                sem))
    for c in copies:
        c.wait()

    lane = lax.iota(jnp.int32, L)
    tr_base = lane * L

    def group(g, acc):
        e0 = g * L
        for j in range(L):
            e = e0 + j
            prod = rows_v[e] * rows_v[BPW + e] * rows_v[2 * BPW + e]
            plsc.store_scatter(tmat_v, [tr_base + j], prod)
        score = tmat_v[pl.ds(0, L)]
        for d in range(1, D):
            score = score + tmat_v[pl.ds(d * L, L)]
        p = jnp.clip(1.0 - jnp.exp(-score), 1e-05, 1.0)
        reward_v[pl.ds(e0, L)] = 1.0 - p
        lbl = label_v[pl.ds(e0, L)]
        return acc + (lbl * p + (1.0 - lbl) * (1.0 - p))

    acc = lax.fori_loop(0, GROUPS, group, jnp.zeros((L,), jnp.float32))

    parts_v[...] = acc
    pltpu.sync_copy(reward_v, reward_hbm.at[wid])
    pltpu.sync_copy(parts_v, parts_hbm.at[wid])


@functools.partial(
    pl.kernel,
    out_type=[jax.ShapeDtypeStruct((NW, BPW), jnp.float32),
              jax.ShapeDtypeStruct((NW, L), jnp.float32)],
    mesh=plsc.VectorSubcoreMesh(core_axis_name="c", subcore_axis_name="s"),
    compiler_params=pltpu.CompilerParams(needs_layout_passes=False,
                                         use_tc_tiling_on_sc=False),
    scratch_types=[
        pltpu.VMEM((MOTIF, NCH, CHUNK), jnp.int32),
        pltpu.VMEM((MOTIF * BPW, D), jnp.float32),
        pltpu.VMEM((BPW,), jnp.float32),
        pltpu.VMEM((BPW,), jnp.float32),
        pltpu.VMEM((D * L,), jnp.float32),
        pltpu.VMEM((L,), jnp.float32),
        pltpu.SemaphoreType.DMA,
    ],
)
def _sc_kernel(motifs_hbm, label_hbm, table_hbm, reward_hbm, parts_hbm,
               idx_v, rows_v, label_v, reward_v, tmat_v, parts_v, sem):
    _sc_body(motifs_hbm, label_hbm, table_hbm, reward_hbm, parts_hbm,
             idx_v, rows_v, label_v, reward_v, tmat_v, parts_v, sem)


def _loss_body(parts_ref, out_ref):
    out_ref[0, 0] = -jnp.sum(parts_ref[...])


def _loss_finish(parts):
    return pl.pallas_call(
        _loss_body,
        out_shape=jax.ShapeDtypeStruct((1, 1), jnp.float32),
        out_specs=pl.BlockSpec(memory_space=pltpu.SMEM),
    )(parts)


@jax.jit
def kernel(embedding_matrix, motifs, label):
    tail = jnp.zeros((8 * 8, D), jnp.float32)
    tail = tail.at[:N_NODES - TAIL0].set(embedding_matrix[TAIL0:])
    lin = _detile_sc(embedding_matrix.T, tail.reshape(8, 128))
    table = lin.reshape(LIN_ROWS * 8, D)
    motifs_w = motifs.T.astype(jnp.int32).reshape(MOTIF, NW, NCH, CHUNK)
    label_w = label.reshape(NW, BPW)
    reward_w, parts = _sc_kernel(motifs_w, label_w, table)
    loss = _loss_finish(parts)[0, 0]
    return (loss, reward_w.reshape(B))


# speedup vs baseline: 1.4874x; 1.3079x over previous
import functools

import jax
import jax.numpy as jnp
from jax import lax
from jax.experimental import pallas as pl
from jax.experimental.pallas import tpu as pltpu
from jax.experimental.pallas import tpu_sc as plsc

NC = 2
NS = 16
NW = NC * NS
L = 16

N_NODES = 100000
D = 16
B = 16384
MOTIF = 3

BPW = B // NW
CHUNK = 128
NCH = BPW // CHUNK
GROUPS = BPW // L

FULL_TC = N_NODES // 128
TAIL0 = FULL_TC * 128
LIN_ROWS = 12504
CPW = 25
CLAST = FULL_TC - 31 * CPW


SK = 9


def _transpose_cols(buf_v, lin_v, lane, ncols):
    skew = (SK * lane) & 127

    def one_col(k, carry):
        rows_k = 16 * k + lane
        base_k = jnp.full((L,), 2048 * k, jnp.int32)
        for l in range(128):
            t = (skew + l) & 127
            row = plsc.load_gather(buf_v, [rows_k, t])
            q = t * 16 + lane
            plsc.store_scatter(
                lin_v, [(base_k + q) >> 7, q & 127], row)
        return carry

    lax.fori_loop(0, ncols, one_col, 0)


def _stage_planes(emb_hbm, buf_v, col0, ncols, sem):
    copies = []
    for k in range(ncols):
        for p in range(2):
            copies.append(pltpu.async_copy(
                emb_hbm.at[pl.ds(8 * p, 8), pl.ds(col0 + 128 * k, 128)],
                buf_v.at[pl.ds(16 * k + 8 * p, 8)],
                sem))
    for c in copies:
        c.wait()


def _detile_body(emb_hbm, tail_hbm, out_hbm, buf_v, lin_v, tail_v, sem):
    wid = lax.axis_index("s") * NC + lax.axis_index("c")
    lane = lax.iota(jnp.int32, L)
    col0 = wid * (CPW * 128)

    @pl.when(wid < 31)
    def _():
        _stage_planes(emb_hbm, buf_v, col0, CPW, sem)
        _transpose_cols(buf_v, lin_v, lane, CPW)
        pltpu.sync_copy(lin_v, out_hbm.at[pl.ds(wid * (CPW * 16), CPW * 16)])

    @pl.when(wid == 31)
    def _():
        _stage_planes(emb_hbm, buf_v, col0, CLAST, sem)
        _transpose_cols(buf_v, lin_v, lane, CLAST)
        pltpu.sync_copy(lin_v.at[pl.ds(0, CLAST * 16)],
                        out_hbm.at[pl.ds(31 * (CPW * 16), CLAST * 16)])
        pltpu.sync_copy(tail_hbm, tail_v)
        pltpu.sync_copy(tail_v, out_hbm.at[pl.ds(LIN_ROWS - 8, 8)])


@functools.partial(
    pl.kernel,
    out_type=jax.ShapeDtypeStruct((LIN_ROWS, 128), jnp.float32),
    mesh=plsc.VectorSubcoreMesh(core_axis_name="c", subcore_axis_name="s"),
    compiler_params=pltpu.CompilerParams(needs_layout_passes=False,
                                         use_tc_tiling_on_sc=True),
    scratch_types=[
        pltpu.VMEM((CPW * 16, 128), jnp.float32),
        pltpu.VMEM((CPW * 16, 128), jnp.float32),
        pltpu.VMEM((8, 128), jnp.float32),
        pltpu.SemaphoreType.DMA,
    ],
)
def _detile_sc(emb_hbm, tail_hbm, out_hbm, buf_v, lin_v, tail_v, sem):
    _detile_body(emb_hbm, tail_hbm, out_hbm, buf_v, lin_v, tail_v, sem)


def _sc_body(motifs_hbm, label_hbm, table_hbm, reward_hbm, parts_hbm,
             idx_v, rows_v, label_v, reward_v, tmat_v, parts_v, sem):
    wid = lax.axis_index("s") * NC + lax.axis_index("c")

    pltpu.sync_copy(motifs_hbm.at[:, wid], idx_v)
    pltpu.sync_copy(label_hbm.at[wid], label_v)

    copies = []
    for m in range(MOTIF):
        for j in range(NCH):
            copies.append(pltpu.async_copy(
                table_hbm.at[idx_v.at[m, j]],
                rows_v.at[pl.ds((m * NCH + j) * CHUNK, CHUNK)],
                sem))
    for c in copies:
        c.wait()

    lane = lax.iota(jnp.int32, L)
    tr_base = lane * L

    def group(g, acc):
        e0 = g * L
        for j in range(L):
            e = e0 + j
            prod = rows_v[e] * rows_v[BPW + e] * rows_v[2 * BPW + e]
            plsc.store_scatter(tmat_v, [tr_base + j], prod)
        score = tmat_v[pl.ds(0, L)]
        for d in range(1, D):
            score = score + tmat_v[pl.ds(d * L, L)]
        p = jnp.clip(1.0 - jnp.exp(-score), 1e-05, 1.0)
        reward_v[pl.ds(e0, L)] = 1.0 - p
        lbl = label_v[pl.ds(e0, L)]
        return acc + (lbl * p + (1.0 - lbl) * (1.0 - p))

    acc = lax.fori_loop(0, GROUPS, group, jnp.zeros((L,), jnp.float32))

    parts_v[...] = acc
    pltpu.sync_copy(reward_v, reward_hbm.at[wid])
    pltpu.sync_copy(parts_v, parts_hbm.at[wid])


@functools.partial(
    pl.kernel,
    out_type=[jax.ShapeDtypeStruct((NW, BPW), jnp.float32),
              jax.ShapeDtypeStruct((NW, L), jnp.float32)],
    mesh=plsc.VectorSubcoreMesh(core_axis_name="c", subcore_axis_name="s"),
    compiler_params=pltpu.CompilerParams(needs_layout_passes=False,
                                         use_tc_tiling_on_sc=False),
    scratch_types=[
        pltpu.VMEM((MOTIF, NCH, CHUNK), jnp.int32),
        pltpu.VMEM((MOTIF * BPW, D), jnp.float32),
        pltpu.VMEM((BPW,), jnp.float32),
        pltpu.VMEM((BPW,), jnp.float32),
        pltpu.VMEM((D * L,), jnp.float32),
        pltpu.VMEM((L,), jnp.float32),
        pltpu.SemaphoreType.DMA,
    ],
)
def _sc_kernel(motifs_hbm, label_hbm, table_hbm, reward_hbm, parts_hbm,
               idx_v, rows_v, label_v, reward_v, tmat_v, parts_v, sem):
    _sc_body(motifs_hbm, label_hbm, table_hbm, reward_hbm, parts_hbm,
             idx_v, rows_v, label_v, reward_v, tmat_v, parts_v, sem)


def _loss_body(parts_ref, out_ref):
    out_ref[0, 0] = -jnp.sum(parts_ref[...])


def _loss_finish(parts):
    return pl.pallas_call(
        _loss_body,
        out_shape=jax.ShapeDtypeStruct((1, 1), jnp.float32),
        out_specs=pl.BlockSpec(memory_space=pltpu.SMEM),
    )(parts)


@jax.jit
def kernel(embedding_matrix, motifs, label):
    tail = jnp.zeros((8 * 8, D), jnp.float32)
    tail = tail.at[:N_NODES - TAIL0].set(embedding_matrix[TAIL0:])
    lin = _detile_sc(embedding_matrix.T, tail.reshape(8, 128))
    table = lin.reshape(LIN_ROWS * 8, D)
    motifs_w = motifs.T.astype(jnp.int32).reshape(MOTIF, NW, NCH, CHUNK)
    label_w = label.reshape(NW, BPW)
    reward_w, parts = _sc_kernel(motifs_w, label_w, table)
    loss = _loss_finish(parts)[0, 0]
    return (loss, reward_w.reshape(B))


# speedup vs baseline: 1.9441x; 1.3070x over previous
import functools

import jax
import jax.numpy as jnp
from jax import lax
from jax.experimental import pallas as pl
from jax.experimental.pallas import tpu as pltpu
from jax.experimental.pallas import tpu_sc as plsc

NC = 2
NS = 16
NW = NC * NS
L = 16

N_NODES = 100000
D = 16
B = 16384
MOTIF = 3

BPW = B // NW
CHUNK = 128
NCH = BPW // CHUNK
GROUPS = BPW // L

FULL_TC = N_NODES // 128
TAIL0 = FULL_TC * 128
LIN_ROWS = 12504
CPW = 25
CLAST = FULL_TC - 31 * CPW


SK = 1


def _transpose_cols(buf_v, lin_v, lane, ncols):
    skew = (SK * lane) & 127

    def one_col(k, carry):
        rows_k = 16 * k + lane
        base_k = jnp.full((L,), 2048 * k, jnp.int32)
        for l in range(128):
            t = (skew + l) & 127
            row = plsc.load_gather(buf_v, [rows_k, t])
            q = t * 16 + lane
            plsc.store_scatter(
                lin_v, [(base_k + q) >> 7, q & 127], row)
        return carry

    lax.fori_loop(0, ncols, one_col, 0)


def _stage_planes(emb_hbm, buf_v, col0, ncols, sem):
    copies = []
    for k in range(ncols):
        for p in range(2):
            copies.append(pltpu.async_copy(
                emb_hbm.at[pl.ds(8 * p, 8), pl.ds(col0 + 128 * k, 128)],
                buf_v.at[pl.ds(16 * k + 8 * p, 8)],
                sem))
    for c in copies:
        c.wait()


def _detile_body(emb_hbm, tail_hbm, out_hbm, buf_v, lin_v, tail_v, sem):
    wid = lax.axis_index("s") * NC + lax.axis_index("c")
    lane = lax.iota(jnp.int32, L)
    col0 = wid * (CPW * 128)

    @pl.when(wid < 31)
    def _():
        _stage_planes(emb_hbm, buf_v, col0, CPW, sem)
        _transpose_cols(buf_v, lin_v, lane, CPW)
        pltpu.sync_copy(lin_v, out_hbm.at[pl.ds(wid * (CPW * 16), CPW * 16)])

    @pl.when(wid == 31)
    def _():
        _stage_planes(emb_hbm, buf_v, col0, CLAST, sem)
        _transpose_cols(buf_v, lin_v, lane, CLAST)
        pltpu.sync_copy(lin_v.at[pl.ds(0, CLAST * 16)],
                        out_hbm.at[pl.ds(31 * (CPW * 16), CLAST * 16)])
        pltpu.sync_copy(tail_hbm, tail_v)
        pltpu.sync_copy(tail_v, out_hbm.at[pl.ds(LIN_ROWS - 8, 8)])


@functools.partial(
    pl.kernel,
    out_type=jax.ShapeDtypeStruct((LIN_ROWS, 128), jnp.float32),
    mesh=plsc.VectorSubcoreMesh(core_axis_name="c", subcore_axis_name="s"),
    compiler_params=pltpu.CompilerParams(needs_layout_passes=False,
                                         use_tc_tiling_on_sc=True),
    scratch_types=[
        pltpu.VMEM((CPW * 16, 128), jnp.float32),
        pltpu.VMEM((CPW * 16, 128), jnp.float32),
        pltpu.VMEM((8, 128), jnp.float32),
        pltpu.SemaphoreType.DMA,
    ],
)
def _detile_sc(emb_hbm, tail_hbm, out_hbm, buf_v, lin_v, tail_v, sem):
    _detile_body(emb_hbm, tail_hbm, out_hbm, buf_v, lin_v, tail_v, sem)


def _sc_body(motifs_hbm, label_hbm, table_hbm, reward_hbm, parts_hbm,
             idx_v, rows_v, label_v, reward_v, tmat_v, parts_v, sem):
    wid = lax.axis_index("s") * NC + lax.axis_index("c")

    pltpu.sync_copy(motifs_hbm.at[:, wid], idx_v)
    pltpu.sync_copy(label_hbm.at[wid], label_v)

    copies = []
    for m in range(MOTIF):
        for j in range(NCH):
            copies.append(pltpu.async_copy(
                table_hbm.at[idx_v.at[m, j]],
                rows_v.at[pl.ds((m * NCH + j) * CHUNK, CHUNK)],
                sem))
    for c in copies:
        c.wait()

    lane = lax.iota(jnp.int32, L)
    tr_base = lane * L

    def group(g, acc):
        e0 = g * L
        for j in range(L):
            e = e0 + j
            prod = rows_v[e] * rows_v[BPW + e] * rows_v[2 * BPW + e]
            plsc.store_scatter(tmat_v, [tr_base + j], prod)
        score = tmat_v[pl.ds(0, L)]
        for d in range(1, D):
            score = score + tmat_v[pl.ds(d * L, L)]
        p = jnp.clip(1.0 - jnp.exp(-score), 1e-05, 1.0)
        reward_v[pl.ds(e0, L)] = 1.0 - p
        lbl = label_v[pl.ds(e0, L)]
        return acc + (lbl * p + (1.0 - lbl) * (1.0 - p))

    acc = lax.fori_loop(0, GROUPS, group, jnp.zeros((L,), jnp.float32))

    parts_v[...] = acc
    pltpu.sync_copy(reward_v, reward_hbm.at[wid])
    pltpu.sync_copy(parts_v, parts_hbm.at[wid])


@functools.partial(
    pl.kernel,
    out_type=[jax.ShapeDtypeStruct((NW, BPW), jnp.float32),
              jax.ShapeDtypeStruct((NW, L), jnp.float32)],
    mesh=plsc.VectorSubcoreMesh(core_axis_name="c", subcore_axis_name="s"),
    compiler_params=pltpu.CompilerParams(needs_layout_passes=False,
                                         use_tc_tiling_on_sc=False),
    scratch_types=[
        pltpu.VMEM((MOTIF, NCH, CHUNK), jnp.int32),
        pltpu.VMEM((MOTIF * BPW, D), jnp.float32),
        pltpu.VMEM((BPW,), jnp.float32),
        pltpu.VMEM((BPW,), jnp.float32),
        pltpu.VMEM((D * L,), jnp.float32),
        pltpu.VMEM((L,), jnp.float32),
        pltpu.SemaphoreType.DMA,
    ],
)
def _sc_kernel(motifs_hbm, label_hbm, table_hbm, reward_hbm, parts_hbm,
               idx_v, rows_v, label_v, reward_v, tmat_v, parts_v, sem):
    _sc_body(motifs_hbm, label_hbm, table_hbm, reward_hbm, parts_hbm,
             idx_v, rows_v, label_v, reward_v, tmat_v, parts_v, sem)


def _loss_body(parts_ref, out_ref):
    out_ref[0, 0] = -jnp.sum(parts_ref[...])


def _loss_finish(parts):
    return pl.pallas_call(
        _loss_body,
        out_shape=jax.ShapeDtypeStruct((1, 1), jnp.float32),
        out_specs=pl.BlockSpec(memory_space=pltpu.SMEM),
    )(parts)


@jax.jit
def kernel(embedding_matrix, motifs, label):
    tail = jnp.zeros((8 * 8, D), jnp.float32)
    tail = tail.at[:N_NODES - TAIL0].set(embedding_matrix[TAIL0:])
    lin = _detile_sc(embedding_matrix.T, tail.reshape(8, 128))
    table = lin.reshape(LIN_ROWS * 8, D)
    motifs_w = motifs.T.astype(jnp.int32).reshape(MOTIF, NW, NCH, CHUNK)
    label_w = label.reshape(NW, BPW)
    reward_w, parts = _sc_kernel(motifs_w, label_w, table)
    loss = _loss_finish(parts)[0, 0]
    return (loss, reward_w.reshape(B))


# speedup vs baseline: 1.9457x; 1.0009x over previous
import functools

import jax
import jax.numpy as jnp
from jax import lax
from jax.experimental import pallas as pl
from jax.experimental.pallas import tpu as pltpu
from jax.experimental.pallas import tpu_sc as plsc

NC = 2
NS = 16
NW = NC * NS
L = 16

N_NODES = 100000
D = 16
B = 16384
MOTIF = 3

BPW = B // NW
CHUNK = 128
NCH = BPW // CHUNK
GROUPS = BPW // L

FULL_TC = N_NODES // 128
TAIL0 = FULL_TC * 128
LIN_ROWS = 12504
CPW = 25
CLAST = FULL_TC - 31 * CPW


SK = 1


def _transpose_cols(buf_v, lin_v, lane, ncols):
    skew = (SK * lane) & 127

    def one_col(k, carry):
        rows_k = 16 * k + lane
        base_k = jnp.full((L,), 2048 * k, jnp.int32)
        for l in range(128):
            t = (skew + l) & 127
            row = plsc.load_gather(buf_v, [rows_k, t])
            q = t * 16 + lane
            plsc.store_scatter(
                lin_v, [(base_k + q) >> 7, q & 127], row)
        return carry

    lax.fori_loop(0, ncols, one_col, 0)


def _stage_planes(emb_hbm, buf_v, col0, ncols, sem):
    copies = []
    for k in range(ncols):
        for p in range(2):
            copies.append(pltpu.async_copy(
                emb_hbm.at[pl.ds(8 * p, 8), pl.ds(col0 + 128 * k, 128)],
                buf_v.at[pl.ds(16 * k + 8 * p, 8)],
                sem))
    for c in copies:
        c.wait()


def _detile_body(emb_hbm, tail_hbm, out_hbm, buf_v, lin_v, tail_v, sem):
    wid = lax.axis_index("s") * NC + lax.axis_index("c")
    lane = lax.iota(jnp.int32, L)
    col0 = wid * (CPW * 128)

    @pl.when(wid < 31)
    def _():
        _stage_planes(emb_hbm, buf_v, col0, CPW, sem)
        _transpose_cols(buf_v, lin_v, lane, CPW)
        pltpu.sync_copy(lin_v, out_hbm.at[pl.ds(wid * (CPW * 16), CPW * 16)])

    @pl.when(wid == 31)
    def _():
        _stage_planes(emb_hbm, buf_v, col0, CLAST, sem)
        _transpose_cols(buf_v, lin_v, lane, CLAST)
        pltpu.sync_copy(lin_v.at[pl.ds(0, CLAST * 16)],
                        out_hbm.at[pl.ds(31 * (CPW * 16), CLAST * 16)])
        pltpu.sync_copy(tail_hbm, tail_v)
        pltpu.sync_copy(tail_v, out_hbm.at[pl.ds(LIN_ROWS - 8, 8)])


@functools.partial(
    pl.kernel,
    out_type=jax.ShapeDtypeStruct((LIN_ROWS, 128), jnp.float32),
    mesh=plsc.VectorSubcoreMesh(core_axis_name="c", subcore_axis_name="s"),
    compiler_params=pltpu.CompilerParams(needs_layout_passes=False,
                                         use_tc_tiling_on_sc=True),
    scratch_types=[
        pltpu.VMEM((CPW * 16, 128), jnp.float32),
        pltpu.VMEM((CPW * 16, 128), jnp.float32),
        pltpu.VMEM((8, 128), jnp.float32),
        pltpu.SemaphoreType.DMA,
    ],
)
def _detile_sc(emb_hbm, tail_hbm, out_hbm, buf_v, lin_v, tail_v, sem):
    _detile_body(emb_hbm, tail_hbm, out_hbm, buf_v, lin_v, tail_v, sem)


def _sc_body(motifs_hbm, label_hbm, table_hbm, reward_hbm, parts_hbm,
             idx_v, rows_v, label_v, reward_v, tmat_v, parts_v, sem):
    wid = lax.axis_index("s") * NC + lax.axis_index("c")

    pltpu.sync_copy(motifs_hbm.at[:, wid], idx_v)
    pltpu.sync_copy(label_hbm.at[wid], label_v)

    copies = []
    for m in range(MOTIF):
        for j in range(NCH):
            copies.append(pltpu.async_copy(
                table_hbm.at[idx_v.at[m, j]],
                rows_v.at[pl.ds((m * NCH + j) * CHUNK, CHUNK)],
                sem))
    for c in copies:
        c.wait()

    lane = lax.iota(jnp.int32, L)
    tr_base = lane * (L + 1)

    def group(g, acc):
        e0 = g * L
        for j in range(L):
            e = e0 + j
            prod = rows_v[e] * rows_v[BPW + e] * rows_v[2 * BPW + e]
            plsc.store_scatter(tmat_v, [tr_base + j], prod)
        score = tmat_v[pl.ds(0, L)]
        for d in range(1, D):
            score = score + tmat_v[pl.ds(d * (L + 1), L)]
        p = jnp.clip(1.0 - jnp.exp(-score), 1e-05, 1.0)
        reward_v[pl.ds(e0, L)] = 1.0 - p
        lbl = label_v[pl.ds(e0, L)]
        return acc + (lbl * p + (1.0 - lbl) * (1.0 - p))

    acc = lax.fori_loop(0, GROUPS, group, jnp.zeros((L,), jnp.float32))

    parts_v[...] = acc
    pltpu.sync_copy(reward_v, reward_hbm.at[wid])
    pltpu.sync_copy(parts_v, parts_hbm.at[wid])


@functools.partial(
    pl.kernel,
    out_type=[jax.ShapeDtypeStruct((NW, BPW), jnp.float32),
              jax.ShapeDtypeStruct((NW, L), jnp.float32)],
    mesh=plsc.VectorSubcoreMesh(core_axis_name="c", subcore_axis_name="s"),
    compiler_params=pltpu.CompilerParams(needs_layout_passes=False,
                                         use_tc_tiling_on_sc=False),
    scratch_types=[
        pltpu.VMEM((MOTIF, NCH, CHUNK), jnp.int32),
        pltpu.VMEM((MOTIF * BPW, D), jnp.float32),
        pltpu.VMEM((BPW,), jnp.float32),
        pltpu.VMEM((BPW,), jnp.float32),
        pltpu.VMEM((D * (L + 1),), jnp.float32),
        pltpu.VMEM((L,), jnp.float32),
        pltpu.SemaphoreType.DMA,
    ],
)
def _sc_kernel(motifs_hbm, label_hbm, table_hbm, reward_hbm, parts_hbm,
               idx_v, rows_v, label_v, reward_v, tmat_v, parts_v, sem):
    _sc_body(motifs_hbm, label_hbm, table_hbm, reward_hbm, parts_hbm,
             idx_v, rows_v, label_v, reward_v, tmat_v, parts_v, sem)


def _loss_body(parts_ref, out_ref):
    out_ref[0, 0] = -jnp.sum(parts_ref[...])


def _loss_finish(parts):
    return pl.pallas_call(
        _loss_body,
        out_shape=jax.ShapeDtypeStruct((1, 1), jnp.float32),
        out_specs=pl.BlockSpec(memory_space=pltpu.SMEM),
    )(parts)


@jax.jit
def kernel(embedding_matrix, motifs, label):
    tail = jnp.zeros((8 * 8, D), jnp.float32)
    tail = tail.at[:N_NODES - TAIL0].set(embedding_matrix[TAIL0:])
    lin = _detile_sc(embedding_matrix.T, tail.reshape(8, 128))
    table = lin.reshape(LIN_ROWS * 8, D)
    motifs_w = motifs.T.astype(jnp.int32).reshape(MOTIF, NW, NCH, CHUNK)
    label_w = label.reshape(NW, BPW)
    reward_w, parts = _sc_kernel(motifs_w, label_w, table)
    loss = _loss_finish(parts)[0, 0]
    return (loss, reward_w.reshape(B))
